# baseline (speedup 1.0000x reference)
import functools

import jax
import jax.numpy as jnp
import numpy as np
from jax.experimental import pallas as pl
from jax.experimental.pallas import tpu as pltpu

_DIM = 768
_NH = 12
_HD = 64
_NE = 8
_TOPK = 2
_HID = 2048
_EPS = 1e-6


def _qkv_body(x_ref, w_ref, g_ref, cos_ref, sin_ref, o_ref):
    xb = x_ref[...]
    ms = jnp.mean(xb * xb, axis=1, keepdims=True)
    xn = xb * jax.lax.rsqrt(ms + _EPS) * g_ref[...]
    qkv = jnp.dot(xn, w_ref[...], preferred_element_type=jnp.float32)
    cos = cos_ref[...]
    sin = sin_ref[...]
    half = _HD // 2
    for part in range(2):
        for h in range(_NH):
            c0 = part * _DIM + h * _HD
            a = qkv[:, c0:c0 + half]
            b = qkv[:, c0 + half:c0 + _HD]
            o_ref[part * _NH + h, :, :half] = a * cos - b * sin
            o_ref[part * _NH + h, :, half:] = a * sin + b * cos
    for h in range(_NH):
        c0 = 2 * _DIM + h * _HD
        o_ref[2 * _NH + h, :, :] = qkv[:, c0:c0 + _HD]


def _attn_body(q_ref, k_ref, v_ref, o_ref):
    q = q_ref[0]
    k = k_ref[0]
    v = v_ref[0]
    s = jax.lax.dot_general(q, k, (((1,), (1,)), ((), ())),
                            preferred_element_type=jnp.float32)
    s = s * (_HD ** -0.5)
    s = s - jnp.max(s, axis=1, keepdims=True)
    p = jnp.exp(s)
    p = p / jnp.sum(p, axis=1, keepdims=True)
    o_ref[0] = jnp.dot(p, v, preferred_element_type=jnp.float32)


def _router_body(attn_ref, x_ref, wp_ref, g2_ref, wg_ref,
                 x2_ref, xn2_ref, ti_ref, tw_ref, aux_ref):
    proj = jnp.dot(attn_ref[0], wp_ref[0], preferred_element_type=jnp.float32)
    for h in range(1, _NH):
        proj = proj + jnp.dot(attn_ref[h], wp_ref[h],
                              preferred_element_type=jnp.float32)
    x2 = x_ref[...] + proj
    x2_ref[...] = x2
    ms = jnp.mean(x2 * x2, axis=1, keepdims=True)
    xn2 = x2 * jax.lax.rsqrt(ms + _EPS) * g2_ref[...]
    xn2_ref[...] = xn2
    gate = jnp.dot(xn2, wg_ref[...], preferred_element_type=jnp.float32)
    lanes = jax.lax.broadcasted_iota(jnp.int32, gate.shape, 1)
    m1 = jnp.max(gate, axis=1, keepdims=True)
    eq1 = gate == m1
    i1 = jnp.min(jnp.where(eq1, lanes, _NE), axis=1, keepdims=True)
    oh1 = (lanes == i1)
    masked = jnp.where(oh1, -jnp.inf, gate)
    m2 = jnp.max(masked, axis=1, keepdims=True)
    eq2 = masked == m2
    i2 = jnp.min(jnp.where(eq2, lanes, _NE), axis=1, keepdims=True)
    oh2 = (lanes == i2)
    e2 = jnp.exp(m2 - m1)
    w1 = 1.0 / (1.0 + e2)
    w2 = e2 * w1
    ti_ref[...] = jnp.where(lanes == 0, i1, 0) + jnp.where(lanes == 1, i2, 0)
    tw_ref[...] = jnp.where(lanes == 0, w1, 0.0) + jnp.where(lanes == 1, w2,
                                                             0.0)
    pm = jnp.exp(gate - m1)
    p = pm / jnp.sum(pm, axis=1, keepdims=True)
    usage = jnp.mean(p, axis=0, keepdims=True)
    aux_ref[0, 0] = _NE * jnp.sum(usage * usage)


_TB = 256
_NS = 2 * _TOPK * 2048 // _TB + _NE - 1


def _gmoe_body(meta_ref, xs_ref, ws_ref, w1_ref, w3_ref, w2_ref, o_ref):
    s = pl.program_id(0)
    tile = meta_ref[1, s]
    vs = meta_ref[2, s]
    ve = meta_ref[3, s]
    first = meta_ref[4, s]
    xb = xs_ref[...]
    h1 = jax.lax.dot_general(xb, w1_ref[0], (((1,), (1,)), ((), ())),
                             preferred_element_type=jnp.float32)
    h3 = jax.lax.dot_general(xb, w3_ref[0], (((1,), (1,)), ((), ())),
                             preferred_element_type=jnp.float32)
    h = h1 * jax.lax.logistic(h1) * h3
    y = jax.lax.dot_general(h, w2_ref[0], (((1,), (1,)), ((), ())),
                            preferred_element_type=jnp.float32)
    rg = tile * _TB + jax.lax.broadcasted_iota(jnp.int32, (_TB, 1), 0)
    wm = jnp.where((rg >= vs) & (rg < ve), ws_ref[0], 0.0)
    contrib = wm * y

    @pl.when(first == 1)
    def _init():
        o_ref[...] = contrib

    @pl.when(first == 0)
    def _acc():
        o_ref[...] = o_ref[...] + contrib


def _comb_body(x2_ref, ya_ref, yb_ref, o_ref):
    o_ref[...] = x2_ref[...] + ya_ref[...] + yb_ref[...]


def _sc_gather_rows(table, idx):
    from jax.experimental.pallas import tpu_sc as plsc

    n, d = idx.shape[0], table.shape[1]
    nw = 32
    bpw = n // nw
    mesh = plsc.VectorSubcoreMesh(core_axis_name="c", subcore_axis_name="s")

    @functools.partial(
        pl.kernel, mesh=mesh,
        out_type=jax.ShapeDtypeStruct((n, d), jnp.float32),
        scratch_types=[
            pltpu.VMEM((bpw,), jnp.int32),
            pltpu.VMEM((bpw, d), jnp.float32),
            pltpu.SemaphoreType.DMA,
        ],
    )
    def k(table_hbm, idx_hbm, out_hbm, idx_v, rows_v, sem):
        wid = jax.lax.axis_index("s") * 2 + jax.lax.axis_index("c")
        base = wid * bpw
        pltpu.sync_copy(idx_hbm.at[pl.ds(base, bpw)], idx_v)
        pltpu.async_copy(table_hbm.at[idx_v], rows_v, sem).wait()
        pltpu.sync_copy(rows_v, out_hbm.at[pl.ds(base, bpw)])

    return k(table, idx)


def kernel(x, Wqkv, Wproj, Wg, W1, W2, W3, g1, g2):
    Bb, L, D = x.shape
    xf = x.reshape(L, D)

    perm = np.arange(3 * _DIM)
    for part in range(2):
        for h in range(_NH):
            base = part * _DIM + h * _HD
            perm[base:base + _HD] = np.concatenate(
                [np.arange(base, base + _HD, 2),
                 np.arange(base + 1, base + _HD, 2)])
    WqkvT = Wqkv[perm].T

    inv_freq = 1.0 / (10000.0 ** (np.arange(0, _HD, 2, dtype=np.float32)
                                  / _HD))
    t = np.arange(L, dtype=np.float32)
    freqs = np.outer(t, inv_freq)
    cos = jnp.asarray(np.cos(freqs))
    sin = jnp.asarray(np.sin(freqs))

    LT_A = 256
    qkv = pl.pallas_call(
        _qkv_body,
        grid=(L // LT_A,),
        in_specs=[
            pl.BlockSpec((LT_A, D), lambda i: (i, 0)),
            pl.BlockSpec((D, 3 * _DIM), lambda i: (0, 0)),
            pl.BlockSpec((1, D), lambda i: (0, 0)),
            pl.BlockSpec((LT_A, _HD // 2), lambda i: (i, 0)),
            pl.BlockSpec((LT_A, _HD // 2), lambda i: (i, 0)),
        ],
        out_specs=pl.BlockSpec((3 * _NH, LT_A, _HD), lambda i: (0, i, 0)),
        out_shape=jax.ShapeDtypeStruct((3 * _NH, L, _HD), jnp.float32),
    )(xf, WqkvT, g1.reshape(1, D), cos, sin)

    attnc = pl.pallas_call(
        _attn_body,
        grid=(_NH,),
        in_specs=[
            pl.BlockSpec((1, L, _HD), lambda h: (h, 0, 0)),
            pl.BlockSpec((1, L, _HD), lambda h: (h + _NH, 0, 0)),
            pl.BlockSpec((1, L, _HD), lambda h: (h + 2 * _NH, 0, 0)),
        ],
        out_specs=pl.BlockSpec((1, L, _HD), lambda h: (h, 0, 0)),
        out_shape=jax.ShapeDtypeStruct((_NH, L, _HD), jnp.float32),
    )(qkv, qkv, qkv)

    x2, xn2, ti, tw, aux = pl.pallas_call(
        _router_body,
        grid=(1,),
        in_specs=[
            pl.BlockSpec((_NH, L, _HD), lambda i: (0, 0, 0)),
            pl.BlockSpec((L, D), lambda i: (0, 0)),
            pl.BlockSpec((_NH, _HD, D), lambda i: (0, 0, 0)),
            pl.BlockSpec((1, D), lambda i: (0, 0)),
            pl.BlockSpec((D, _NE), lambda i: (0, 0)),
        ],
        out_specs=[
            pl.BlockSpec((L, D), lambda i: (0, 0)),
            pl.BlockSpec((L, D), lambda i: (0, 0)),
            pl.BlockSpec((L, _NE), lambda i: (0, 0)),
            pl.BlockSpec((L, _NE), lambda i: (0, 0)),
            pl.BlockSpec(memory_space=pltpu.SMEM),
        ],
        out_shape=[
            jax.ShapeDtypeStruct((L, D), jnp.float32),
            jax.ShapeDtypeStruct((L, D), jnp.float32),
            jax.ShapeDtypeStruct((L, _NE), jnp.int32),
            jax.ShapeDtypeStruct((L, _NE), jnp.float32),
            jax.ShapeDtypeStruct((1, 1), jnp.float32),
        ],
    )(attnc, xf, Wproj.T.reshape(_NH, _HD, D), g2.reshape(1, D), Wg.T)

    NA = _TOPK * L
    ef = ti[:, :_TOPK].reshape(NA)
    wf = tw[:, :_TOPK].reshape(NA)
    order = jnp.argsort(ef, stable=True).astype(jnp.int32)
    tok = (order // _TOPK).astype(jnp.int32)
    wsorted = wf[order]
    inv = jnp.argsort(order).astype(jnp.int32)
    pA = inv[0::_TOPK]
    pB = inv[1::_TOPK]
    counts = jnp.bincount(ef, length=_NE)
    offs = jnp.concatenate([jnp.zeros((1,), jnp.int32),
                            jnp.cumsum(counts).astype(jnp.int32)])
    start_t = offs[:_NE] // _TB
    n_e = jnp.where(counts > 0, -((-offs[1:]) // _TB) - start_t, 0)
    vcum = jnp.concatenate([jnp.zeros((1,), jnp.int32),
                            jnp.cumsum(n_e).astype(jnp.int32)])
    sarr = jnp.arange(_NS, dtype=jnp.int32)
    eid = jnp.searchsorted(vcum[1:], sarr, side="right").astype(jnp.int32)
    active = sarr < vcum[_NE]
    eidc = jnp.minimum(eid, _NE - 1)
    tile = start_t[eidc] + sarr - vcum[eidc]
    vs = jnp.maximum(offs[eidc], tile * _TB)
    ve = jnp.minimum(offs[eidc + 1], (tile + 1) * _TB)
    tile = jnp.where(active, tile, NA // _TB - 1)
    vs = jnp.where(active, vs, 0)
    ve = jnp.where(active, ve, 0)
    first = jnp.where(active & (vs == tile * _TB), 1, 0)
    meta = jnp.stack([eidc, tile, vs, ve, first]).astype(jnp.int32)

    xs = _sc_gather_rows(xn2, tok)

    ys = pl.pallas_call(
        _gmoe_body,
        grid_spec=pltpu.PrefetchScalarGridSpec(
            num_scalar_prefetch=1,
            grid=(_NS,),
            in_specs=[
                pl.BlockSpec((_TB, D), lambda s, m: (m[1, s], 0)),
                pl.BlockSpec((1, _TB, 1), lambda s, m: (m[1, s], 0, 0)),
                pl.BlockSpec((1, _HID, D), lambda s, m: (m[0, s], 0, 0)),
                pl.BlockSpec((1, _HID, D), lambda s, m: (m[0, s], 0, 0)),
                pl.BlockSpec((1, D, _HID), lambda s, m: (m[0, s], 0, 0)),
            ],
            out_specs=pl.BlockSpec((_TB, D), lambda s, m: (m[1, s], 0)),
        ),
        out_shape=jax.ShapeDtypeStruct((NA, D), jnp.float32),
    )(meta, xs, wsorted.reshape(NA // _TB, _TB, 1), W1, W3, W2)

    yA = _sc_gather_rows(ys, pA)
    yB = _sc_gather_rows(ys, pB)

    LT_F = 256
    y = pl.pallas_call(
        _comb_body,
        grid=(L // LT_F,),
        in_specs=[
            pl.BlockSpec((LT_F, D), lambda i: (i, 0)),
            pl.BlockSpec((LT_F, D), lambda i: (i, 0)),
            pl.BlockSpec((LT_F, D), lambda i: (i, 0)),
        ],
        out_specs=pl.BlockSpec((LT_F, D), lambda i: (i, 0)),
        out_shape=jax.ShapeDtypeStruct((L, D), jnp.float32),
    )(x2, yA, yB)

    return (y.reshape(Bb, L, D), aux[0, 0])


# speedup vs baseline: 1.0482x; 1.0482x over previous
import functools

import jax
import jax.numpy as jnp
import numpy as np
from jax.experimental import pallas as pl
from jax.experimental.pallas import tpu as pltpu

_DIM = 768
_NH = 12
_HD = 64
_NE = 8
_TOPK = 2
_HID = 2048
_EPS = 1e-6


def _qkv_body(x_ref, w_ref, g_ref, cos_ref, sin_ref, o_ref):
    xb = x_ref[...]
    ms = jnp.mean(xb * xb, axis=1, keepdims=True)
    xn = xb * jax.lax.rsqrt(ms + _EPS) * g_ref[...]
    qkv = jnp.dot(xn, w_ref[...], preferred_element_type=jnp.float32)
    cos = cos_ref[...]
    sin = sin_ref[...]
    half = _HD // 2
    for part in range(2):
        for h in range(_NH):
            c0 = part * _DIM + h * _HD
            a = qkv[:, c0:c0 + half]
            b = qkv[:, c0 + half:c0 + _HD]
            o_ref[part * _NH + h, :, :half] = a * cos - b * sin
            o_ref[part * _NH + h, :, half:] = a * sin + b * cos
    for h in range(_NH):
        c0 = 2 * _DIM + h * _HD
        o_ref[2 * _NH + h, :, :] = qkv[:, c0:c0 + _HD]


def _attn_body(q_ref, k_ref, v_ref, o_ref):
    q = q_ref[0]
    k = k_ref[0]
    v = v_ref[0]
    s = jax.lax.dot_general(q, k, (((1,), (1,)), ((), ())),
                            preferred_element_type=jnp.float32)
    s = s * (_HD ** -0.5)
    s = s - jnp.max(s, axis=1, keepdims=True)
    p = jnp.exp(s)
    p = p / jnp.sum(p, axis=1, keepdims=True)
    o_ref[0] = jnp.dot(p, v, preferred_element_type=jnp.float32)


def _router_body(attn_ref, x_ref, wp_ref, g2_ref, wg_ref,
                 x2_ref, xn2_ref, ti_ref, tw_ref, aux_ref):
    proj = jnp.dot(attn_ref[0], wp_ref[0], preferred_element_type=jnp.float32)
    for h in range(1, _NH):
        proj = proj + jnp.dot(attn_ref[h], wp_ref[h],
                              preferred_element_type=jnp.float32)
    x2 = x_ref[...] + proj
    x2_ref[...] = x2
    ms = jnp.mean(x2 * x2, axis=1, keepdims=True)
    xn2 = x2 * jax.lax.rsqrt(ms + _EPS) * g2_ref[...]
    xn2_ref[...] = xn2
    gate = jnp.dot(xn2, wg_ref[...], preferred_element_type=jnp.float32)
    lanes = jax.lax.broadcasted_iota(jnp.int32, gate.shape, 1)
    m1 = jnp.max(gate, axis=1, keepdims=True)
    eq1 = gate == m1
    i1 = jnp.min(jnp.where(eq1, lanes, _NE), axis=1, keepdims=True)
    oh1 = (lanes == i1)
    masked = jnp.where(oh1, -jnp.inf, gate)
    m2 = jnp.max(masked, axis=1, keepdims=True)
    eq2 = masked == m2
    i2 = jnp.min(jnp.where(eq2, lanes, _NE), axis=1, keepdims=True)
    oh2 = (lanes == i2)
    e2 = jnp.exp(m2 - m1)
    w1 = 1.0 / (1.0 + e2)
    w2 = e2 * w1
    ti_ref[...] = jnp.where(lanes == 0, i1, 0) + jnp.where(lanes == 1, i2, 0)
    tw_ref[...] = jnp.where(lanes == 0, w1, 0.0) + jnp.where(lanes == 1, w2,
                                                             0.0)
    pm = jnp.exp(gate - m1)
    p = pm / jnp.sum(pm, axis=1, keepdims=True)
    usage = jnp.mean(p, axis=0, keepdims=True)
    aux_ref[0, 0] = _NE * jnp.sum(usage * usage)


_TB = 256
_NS = 2 * _TOPK * 2048 // _TB + _NE - 1


def _gmoe_body(meta_ref, xs_ref, w1_ref, w3_ref, w2_ref, o_ref):
    xb = xs_ref[...]
    h1 = jax.lax.dot_general(xb, w1_ref[0], (((1,), (1,)), ((), ())),
                             preferred_element_type=jnp.float32)
    h3 = jax.lax.dot_general(xb, w3_ref[0], (((1,), (1,)), ((), ())),
                             preferred_element_type=jnp.float32)
    h = h1 * jax.lax.logistic(h1) * h3
    o_ref[...] = jax.lax.dot_general(h, w2_ref[0], (((1,), (1,)), ((), ())),
                                     preferred_element_type=jnp.float32)


def _rank_body(ti_ref, pos_ref, cnt_ref):
    ti = ti_ref[...]
    n = ti.shape[0]
    lanes = jax.lax.broadcasted_iota(jnp.int32, ti.shape, 1)
    tiA = jnp.sum(jnp.where(lanes == 0, ti, 0), axis=1, keepdims=True)
    tiB = jnp.sum(jnp.where(lanes == 1, ti, 0), axis=1, keepdims=True)
    ohA = jnp.where(lanes == tiA, 1.0, 0.0)
    ohB = jnp.where(lanes == tiB, 1.0, 0.0)
    oh = ohA + ohB
    c = oh
    sh = 1
    while sh < n:
        c = c + jnp.concatenate(
            [jnp.zeros((sh, _NE), jnp.float32), c[:n - sh]], axis=0)
        sh *= 2
    exc = c - oh
    counts = c[n - 1:n, :]

    def _lane_excl_cumsum(v):
        o = v
        s = 1
        while s < _NE:
            o = o + jnp.concatenate(
                [jnp.zeros((1, s), jnp.float32), o[:, :_NE - s]], axis=1)
            s *= 2
        return o - v

    offs = _lane_excl_cumsum(counts)
    ends = offs + counts
    start_t = jnp.floor(offs * (1.0 / _TB))
    end_t = jnp.ceil(ends * (1.0 / _TB))
    n_e = jnp.where(counts > 0.0, end_t - start_t, 0.0)
    vcum = _lane_excl_cumsum(n_e)
    bias = _TB * (vcum - start_t)
    base = offs + exc
    posA = jnp.sum(base * ohA, axis=1, keepdims=True).astype(jnp.int32)
    posB = jnp.sum(base * ohB, axis=1, keepdims=True).astype(jnp.int32)
    qosA = jnp.sum((base + bias) * ohA, axis=1,
                   keepdims=True).astype(jnp.int32)
    qosB = jnp.sum((base + bias) * ohB, axis=1,
                   keepdims=True).astype(jnp.int32)
    pos_ref[...] = (jnp.where(lanes == 0, posA, 0)
                    + jnp.where(lanes == 1, posB, 0)
                    + jnp.where(lanes == 2, qosA, 0)
                    + jnp.where(lanes == 3, qosB, 0))
    cnt_ref[...] = counts.astype(jnp.int32)


def _comb_body(x2_ref, ya_ref, yb_ref, tw_ref, o_ref):
    lanes = jax.lax.broadcasted_iota(jnp.int32, tw_ref.shape, 1)
    tw = tw_ref[...]
    wa = jnp.sum(jnp.where(lanes == 0, tw, 0.0), axis=1, keepdims=True)
    wb = jnp.sum(jnp.where(lanes == 1, tw, 0.0), axis=1, keepdims=True)
    o_ref[...] = x2_ref[...] + wa * ya_ref[...] + wb * yb_ref[...]


def _sc_dispatch(table, pA, pB):
    from jax.experimental.pallas import tpu_sc as plsc

    n, d = table.shape
    bpw = n // 32
    mesh = plsc.VectorSubcoreMesh(core_axis_name="c", subcore_axis_name="s")

    @functools.partial(
        pl.kernel, mesh=mesh,
        out_type=jax.ShapeDtypeStruct((2 * n, d), jnp.float32),
        scratch_types=[
            pltpu.VMEM((bpw,), jnp.int32),
            pltpu.VMEM((bpw,), jnp.int32),
            pltpu.VMEM((bpw, d), jnp.float32),
            pltpu.SemaphoreType.DMA,
        ],
    )
    def k(t_hbm, pa_hbm, pb_hbm, out_hbm, ia_v, ib_v, rows_v, sem):
        wid = jax.lax.axis_index("s") * 2 + jax.lax.axis_index("c")
        base = wid * bpw
        pltpu.sync_copy(t_hbm.at[pl.ds(base, bpw)], rows_v)
        pltpu.sync_copy(pa_hbm.at[pl.ds(base, bpw)], ia_v)
        pltpu.sync_copy(pb_hbm.at[pl.ds(base, bpw)], ib_v)
        pltpu.async_copy(rows_v, out_hbm.at[ia_v], sem).wait()
        pltpu.async_copy(rows_v, out_hbm.at[ib_v], sem).wait()

    return k(table, pA, pB)


def _sc_double_gather(table, pA, pB):
    from jax.experimental.pallas import tpu_sc as plsc

    n = pA.shape[0]
    d = table.shape[1]
    bpw = n // 32
    mesh = plsc.VectorSubcoreMesh(core_axis_name="c", subcore_axis_name="s")

    @functools.partial(
        pl.kernel, mesh=mesh,
        out_type=[jax.ShapeDtypeStruct((n, d), jnp.float32),
                  jax.ShapeDtypeStruct((n, d), jnp.float32)],
        scratch_types=[
            pltpu.VMEM((bpw,), jnp.int32),
            pltpu.VMEM((bpw, d), jnp.float32),
            pltpu.SemaphoreType.DMA,
        ],
    )
    def k(t_hbm, pa_hbm, pb_hbm, oa_hbm, ob_hbm, idx_v, rows_v, sem):
        wid = jax.lax.axis_index("s") * 2 + jax.lax.axis_index("c")
        base = wid * bpw
        pltpu.sync_copy(pa_hbm.at[pl.ds(base, bpw)], idx_v)
        pltpu.async_copy(t_hbm.at[idx_v], rows_v, sem).wait()
        pltpu.sync_copy(rows_v, oa_hbm.at[pl.ds(base, bpw)])
        pltpu.sync_copy(pb_hbm.at[pl.ds(base, bpw)], idx_v)
        pltpu.async_copy(t_hbm.at[idx_v], rows_v, sem).wait()
        pltpu.sync_copy(rows_v, ob_hbm.at[pl.ds(base, bpw)])

    return k(table, pA, pB)


def kernel(x, Wqkv, Wproj, Wg, W1, W2, W3, g1, g2):
    Bb, L, D = x.shape
    xf = x.reshape(L, D)

    perm = np.arange(3 * _DIM)
    for part in range(2):
        for h in range(_NH):
            base = part * _DIM + h * _HD
            perm[base:base + _HD] = np.concatenate(
                [np.arange(base, base + _HD, 2),
                 np.arange(base + 1, base + _HD, 2)])
    WqkvT = Wqkv[perm].T

    inv_freq = 1.0 / (10000.0 ** (np.arange(0, _HD, 2, dtype=np.float32)
                                  / _HD))
    t = np.arange(L, dtype=np.float32)
    freqs = np.outer(t, inv_freq)
    cos = jnp.asarray(np.cos(freqs))
    sin = jnp.asarray(np.sin(freqs))

    LT_A = 256
    qkv = pl.pallas_call(
        _qkv_body,
        grid=(L // LT_A,),
        in_specs=[
            pl.BlockSpec((LT_A, D), lambda i: (i, 0)),
            pl.BlockSpec((D, 3 * _DIM), lambda i: (0, 0)),
            pl.BlockSpec((1, D), lambda i: (0, 0)),
            pl.BlockSpec((LT_A, _HD // 2), lambda i: (i, 0)),
            pl.BlockSpec((LT_A, _HD // 2), lambda i: (i, 0)),
        ],
        out_specs=pl.BlockSpec((3 * _NH, LT_A, _HD), lambda i: (0, i, 0)),
        out_shape=jax.ShapeDtypeStruct((3 * _NH, L, _HD), jnp.float32),
    )(xf, WqkvT, g1.reshape(1, D), cos, sin)

    attnc = pl.pallas_call(
        _attn_body,
        grid=(_NH,),
        in_specs=[
            pl.BlockSpec((1, L, _HD), lambda h: (h, 0, 0)),
            pl.BlockSpec((1, L, _HD), lambda h: (h + _NH, 0, 0)),
            pl.BlockSpec((1, L, _HD), lambda h: (h + 2 * _NH, 0, 0)),
        ],
        out_specs=pl.BlockSpec((1, L, _HD), lambda h: (h, 0, 0)),
        out_shape=jax.ShapeDtypeStruct((_NH, L, _HD), jnp.float32),
    )(qkv, qkv, qkv)

    x2, xn2, ti, tw, aux = pl.pallas_call(
        _router_body,
        grid=(1,),
        in_specs=[
            pl.BlockSpec((_NH, L, _HD), lambda i: (0, 0, 0)),
            pl.BlockSpec((L, D), lambda i: (0, 0)),
            pl.BlockSpec((_NH, _HD, D), lambda i: (0, 0, 0)),
            pl.BlockSpec((1, D), lambda i: (0, 0)),
            pl.BlockSpec((D, _NE), lambda i: (0, 0)),
        ],
        out_specs=[
            pl.BlockSpec((L, D), lambda i: (0, 0)),
            pl.BlockSpec((L, D), lambda i: (0, 0)),
            pl.BlockSpec((L, _NE), lambda i: (0, 0)),
            pl.BlockSpec((L, _NE), lambda i: (0, 0)),
            pl.BlockSpec(memory_space=pltpu.SMEM),
        ],
        out_shape=[
            jax.ShapeDtypeStruct((L, D), jnp.float32),
            jax.ShapeDtypeStruct((L, D), jnp.float32),
            jax.ShapeDtypeStruct((L, _NE), jnp.int32),
            jax.ShapeDtypeStruct((L, _NE), jnp.float32),
            jax.ShapeDtypeStruct((1, 1), jnp.float32),
        ],
    )(attnc, xf, Wproj.T.reshape(_NH, _HD, D), g2.reshape(1, D), Wg.T)

    NA = _TOPK * L
    pos, cnt = pl.pallas_call(
        _rank_body,
        grid=(1,),
        in_specs=[pl.BlockSpec((L, _NE), lambda i: (0, 0))],
        out_specs=[
            pl.BlockSpec((L, _NE), lambda i: (0, 0)),
            pl.BlockSpec((1, _NE), lambda i: (0, 0)),
        ],
        out_shape=[
            jax.ShapeDtypeStruct((L, _NE), jnp.int32),
            jax.ShapeDtypeStruct((1, _NE), jnp.int32),
        ],
    )(ti)
    pA = pos[:, 0]
    pB = pos[:, 1]
    pA2 = pos[:, 2]
    pB2 = pos[:, 3]

    counts = cnt[0]
    offs = jnp.concatenate([jnp.zeros((1,), jnp.int32),
                            jnp.cumsum(counts).astype(jnp.int32)])
    start_t = offs[:_NE] // _TB
    n_e = jnp.where(counts > 0, -((-offs[1:]) // _TB) - start_t, 0)
    vcum = jnp.concatenate([jnp.zeros((1,), jnp.int32),
                            jnp.cumsum(n_e).astype(jnp.int32)])
    sarr = jnp.arange(_NS, dtype=jnp.int32)
    eid = jnp.searchsorted(vcum[1:], sarr, side="right").astype(jnp.int32)
    active = sarr < vcum[_NE]
    eidc = jnp.minimum(eid, _NE - 1)
    tile = start_t[eidc] + sarr - vcum[eidc]
    tile = jnp.where(active, tile, NA // _TB - 1)
    meta = jnp.stack([eidc, tile]).astype(jnp.int32)

    xs = _sc_dispatch(xn2, pA, pB)

    ys = pl.pallas_call(
        _gmoe_body,
        grid_spec=pltpu.PrefetchScalarGridSpec(
            num_scalar_prefetch=1,
            grid=(_NS,),
            in_specs=[
                pl.BlockSpec((_TB, D), lambda s, m: (m[1, s], 0)),
                pl.BlockSpec((1, _HID, D), lambda s, m: (m[0, s], 0, 0)),
                pl.BlockSpec((1, _HID, D), lambda s, m: (m[0, s], 0, 0)),
                pl.BlockSpec((1, D, _HID), lambda s, m: (m[0, s], 0, 0)),
            ],
            out_specs=pl.BlockSpec((_TB, D), lambda s, m: (s, 0)),
        ),
        out_shape=jax.ShapeDtypeStruct((_NS * _TB, D), jnp.float32),
    )(meta, xs, W1, W3, W2)

    yA, yB = _sc_double_gather(ys, pA2, pB2)

    LT_F = 256
    y = pl.pallas_call(
        _comb_body,
        grid=(L // LT_F,),
        in_specs=[
            pl.BlockSpec((LT_F, D), lambda i: (i, 0)),
            pl.BlockSpec((LT_F, D), lambda i: (i, 0)),
            pl.BlockSpec((LT_F, D), lambda i: (i, 0)),
            pl.BlockSpec((LT_F, _NE), lambda i: (i, 0)),
        ],
        out_specs=pl.BlockSpec((LT_F, D), lambda i: (i, 0)),
        out_shape=jax.ShapeDtypeStruct((L, D), jnp.float32),
    )(x2, yA, yB, tw)

    return (y.reshape(Bb, L, D), aux[0, 0])
